# baseline (speedup 1.0000x reference)
import functools
import math

import jax
import jax.numpy as jnp
from jax import lax
from jax.experimental import pallas as pl
from jax.experimental.pallas import tpu as pltpu
from jax.experimental.pallas import tpu_sc as plsc

N_NODES = 10000
N_EDGES = 320000
NUM_RBF = 16
NUM_CHANNELS = 128
CUTOFF = 5.0
NUM_SPECIES = 100

NC = 2
NS = 16
NW = NC * NS
EPW = N_EDGES // NW
EB = 80
NB = EPW // EB
RPS = 624


def _emb_body(z_ref, emb_ref, o_ref):
    z = z_ref[...]
    oh = (z == lax.broadcasted_iota(jnp.int32, (z.shape[0], NUM_SPECIES), 1))
    o_ref[...] = jnp.dot(oh.astype(jnp.float32), emb_ref[...],
                         preferred_element_type=jnp.float32)


def _emb_lookup(node_z, emb_table):
    bn = 2000
    return pl.pallas_call(
        _emb_body,
        grid=(N_NODES // bn,),
        in_specs=[
            pl.BlockSpec((bn, 1), lambda i: (i, 0)),
            pl.BlockSpec((NUM_SPECIES, NUM_CHANNELS), lambda i: (0, 0)),
        ],
        out_specs=pl.BlockSpec((bn, NUM_CHANNELS), lambda i: (i, 0)),
        out_shape=jax.ShapeDtypeStruct((N_NODES, NUM_CHANNELS), jnp.float32),
    )(node_z.reshape(-1, 1), emb_table)


def _edge_w_body(eft_ref, ew_ref, wd_ref, bd_ref, o_ref):
    be = o_ref.shape[0]
    w = lax.dot_general(
        eft_ref[...], wd_ref[...],
        dimension_numbers=(((0,), (0,)), ((), ())),
        preferred_element_type=jnp.float32) + bd_ref[...]
    ew = ew_ref[0]
    c = 0.5 * (jnp.cos(ew * (math.pi / CUTOFF)) + 1.0)
    c = jnp.where(ew < CUTOFF, c, 0.0)
    eye = (lax.broadcasted_iota(jnp.int32, (NUM_CHANNELS, NUM_CHANNELS), 0)
           == lax.broadcasted_iota(jnp.int32, (NUM_CHANNELS, NUM_CHANNELS), 1)
           ).astype(jnp.float32)
    ct = lax.dot_general(eye, c, dimension_numbers=(((1,), (1,)), ((), ())),
                         preferred_element_type=jnp.float32)
    for g in range(be // NUM_CHANNELS):
        o_ref[pl.ds(NUM_CHANNELS * g, NUM_CHANNELS), :] = (
            w[NUM_CHANNELS * g:NUM_CHANNELS * (g + 1), :] * ct[:, g:g + 1])


def _edge_w(edge_feats_t, edge_weight, w_dist, b_dist):
    be = 12800
    return pl.pallas_call(
        _edge_w_body,
        grid=(N_EDGES // be,),
        in_specs=[
            pl.BlockSpec((NUM_RBF, be), lambda i: (0, i)),
            pl.BlockSpec((1, be // NUM_CHANNELS, NUM_CHANNELS),
                         lambda i: (i, 0, 0)),
            pl.BlockSpec((NUM_RBF, NUM_CHANNELS), lambda i: (0, 0)),
            pl.BlockSpec((1, NUM_CHANNELS), lambda i: (0, 0)),
        ],
        out_specs=pl.BlockSpec((be, NUM_CHANNELS), lambda i: (i, 0)),
        out_shape=jax.ShapeDtypeStruct((N_EDGES, NUM_CHANNELS), jnp.float32),
    )(edge_feats_t,
      edge_weight.reshape(N_EDGES // be, be // NUM_CHANNELS, NUM_CHANNELS),
      w_dist, b_dist.reshape(1, -1))


NDAT = 2
NIDX = 4
NPIPE = (NB - 1) // NIDX * NIDX


def _sc_body(xnb_hbm, w_hbm, snd_hbm, rcv_hbm, out_hbm,
             s_v, r_v, w_v, x_v, acc, sem_i, sem_g, sem_w, sem_s):
    cid = lax.axis_index("c")
    sid = lax.axis_index("s")

    def _zrow(i, _):
        for j in range(NUM_CHANNELS // 16):
            w_v[0, i, pl.ds(16 * j, 16)] = jnp.zeros((16,), jnp.float32)
        return 0
    lax.fori_loop(0, EB, _zrow, 0)
    row0 = sid * RPS
    for k in range(RPS // EB):
        pltpu.sync_copy(w_v.at[0], acc.at[pl.ds(row0 + k * EB, EB)])
    rem = RPS - (RPS // EB) * EB
    pltpu.sync_copy(w_v.at[0, pl.ds(0, rem)],
                    acc.at[pl.ds(row0 + (RPS // EB) * EB, rem)])
    tail = N_NODES - NS * RPS
    @pl.when(sid == NS - 1)
    def _zero_tail():
        pltpu.sync_copy(w_v.at[0, pl.ds(0, tail)], acc.at[pl.ds(NS * RPS, tail)])
    plsc.subcore_barrier()

    ebase = (cid * NS + sid) * EPW

    def _issue_idx(blk, ki):
        base = ebase + blk * EB
        pltpu.async_copy(snd_hbm.at[pl.ds(base, EB)], s_v.at[ki], sem_i.at[ki])
        pltpu.async_copy(rcv_hbm.at[pl.ds(base, EB)], r_v.at[ki], sem_i.at[ki])

    def _wait_idx(ki):
        pltpu.make_async_copy(snd_hbm.at[pl.ds(0, EB)], s_v.at[ki], sem_i.at[ki]).wait()
        pltpu.make_async_copy(rcv_hbm.at[pl.ds(0, EB)], r_v.at[ki], sem_i.at[ki]).wait()

    def _issue_fetch(blk, kd, ki):
        base = ebase + blk * EB
        pltpu.async_copy(w_hbm.at[pl.ds(base, EB)], w_v.at[kd], sem_w.at[kd])
        pltpu.async_copy(xnb_hbm.at[s_v.at[ki]], x_v.at[kd], sem_g.at[kd])

    def _wait_fetch(kd, ki):
        pltpu.make_async_copy(w_hbm.at[pl.ds(0, EB)], w_v.at[kd], sem_w.at[kd]).wait()
        pltpu.make_async_copy(xnb_hbm.at[s_v.at[ki]], x_v.at[kd], sem_g.at[kd]).wait()

    def _wait_scatter(kd, ki):
        pltpu.make_async_copy(x_v.at[kd], acc.at[r_v.at[ki]], sem_s.at[kd]).wait()

    def _mult(kd, ki):
        def _mrow(q, _):
            for u in range(2):
                i = q * 2 + u
                for jj in range(NUM_CHANNELS // 16):
                    sl = pl.ds(16 * jj, 16)
                    x_v[kd, i, sl] = x_v[kd, i, sl] * w_v[kd, i, sl]
            return 0
        lax.fori_loop(0, EB // 2, _mrow, 0)

    pltpu.sync_copy(snd_hbm.at[pl.ds(ebase, EB)], s_v.at[0])
    pltpu.sync_copy(rcv_hbm.at[pl.ds(ebase, EB)], r_v.at[0])
    _issue_fetch(0, 0, 0)
    _issue_idx(1, 1)
    _issue_idx(2, 2)

    def _outer(m, _):
        for t in range(NIDX):
            j = m * NIDX + t
            kd = t % NDAT
            _wait_fetch(t % NDAT, t)

            @pl.when(j >= 1)
            def _drain_prev():
                _wait_scatter((t + 1) % NDAT, (t + 3) % NIDX)

            _wait_idx((t + 1) % NIDX)
            _issue_fetch(j + 1, (t + 1) % NDAT, (t + 1) % NIDX)
            _mult(kd, t)
            pltpu.async_copy(x_v.at[kd], acc.at[r_v.at[t]], sem_s.at[kd],
                             add=True)

            @pl.when(j + 3 <= NPIPE)
            def _idx_ahead():
                _issue_idx(j + 3, (t + 3) % NIDX)
        return 0

    lax.fori_loop(0, NPIPE // NIDX, _outer, 0)
    _wait_scatter((NPIPE + 1) % NDAT, (NPIPE + 3) % NIDX)
    _wait_fetch(NPIPE % NDAT, NPIPE % NIDX)
    _mult(NPIPE % NDAT, NPIPE % NIDX)
    pltpu.async_copy(x_v.at[NPIPE % NDAT], acc.at[r_v.at[NPIPE % NIDX]],
                     sem_s.at[NPIPE % NDAT], add=True)
    _wait_scatter(NPIPE % NDAT, NPIPE % NIDX)
    plsc.subcore_barrier()
    pltpu.sync_copy(acc.at[pl.ds(row0, RPS)], out_hbm.at[cid, pl.ds(row0, RPS)])
    @pl.when(sid == NS - 1)
    def _out_tail():
        pltpu.sync_copy(acc.at[pl.ds(NS * RPS, tail)],
                        out_hbm.at[cid, pl.ds(NS * RPS, tail)])


def _sc_aggregate(x_nb, w_msg, senders, receivers):
    mesh = plsc.VectorSubcoreMesh(core_axis_name="c", subcore_axis_name="s",
                                  num_cores=NC, num_subcores=NS)
    f = pl.kernel(
        _sc_body,
        out_type=jax.ShapeDtypeStruct((NC, N_NODES, NUM_CHANNELS), jnp.float32),
        mesh=mesh,
        scratch_types=[
            pltpu.VMEM((NIDX, EB), jnp.int32),
            pltpu.VMEM((NIDX, EB), jnp.int32),
            pltpu.VMEM((NDAT, EB, NUM_CHANNELS), jnp.float32),
            pltpu.VMEM((NDAT, EB, NUM_CHANNELS), jnp.float32),
            pltpu.VMEM_SHARED((N_NODES, NUM_CHANNELS), jnp.float32),
            pltpu.SemaphoreType.DMA((NIDX,)),
            pltpu.SemaphoreType.DMA((NDAT,)),
            pltpu.SemaphoreType.DMA((NDAT,)),
            pltpu.SemaphoreType.DMA((NDAT,)),
        ],
    )
    return f(x_nb, w_msg, senders, receivers)


def _combine_body(nf_ref, p_ref, w1_ref, w2_ref, b_ref, o_ref):
    agg = p_ref[0] + p_ref[1]
    o_ref[...] = (jnp.dot(nf_ref[...], w1_ref[...],
                          preferred_element_type=jnp.float32)
                  + jnp.dot(agg, w2_ref[...],
                            preferred_element_type=jnp.float32)
                  + b_ref[...])


def _combine(node_feats, partials, w_comb, b_comb):
    bn = 2000
    return pl.pallas_call(
        _combine_body,
        grid=(N_NODES // bn,),
        in_specs=[
            pl.BlockSpec((bn, NUM_CHANNELS), lambda i: (i, 0)),
            pl.BlockSpec((NC, bn, NUM_CHANNELS), lambda i: (0, i, 0)),
            pl.BlockSpec((NUM_CHANNELS, NUM_CHANNELS), lambda i: (0, 0)),
            pl.BlockSpec((NUM_CHANNELS, NUM_CHANNELS), lambda i: (0, 0)),
            pl.BlockSpec((1, NUM_CHANNELS), lambda i: (0, 0)),
        ],
        out_specs=pl.BlockSpec((bn, NUM_CHANNELS), lambda i: (i, 0)),
        out_shape=jax.ShapeDtypeStruct((N_NODES, NUM_CHANNELS), jnp.float32),
    )(node_feats, partials, w_comb[:NUM_CHANNELS], w_comb[NUM_CHANNELS:],
      b_comb.reshape(1, -1))


def kernel(node_z, node_feats, senders, receivers, edge_weight, edge_feats,
           emb_table, W_dist, b_dist, W_comb, b_comb):
    x_nb = _emb_lookup(node_z.astype(jnp.int32), emb_table)
    w_msg = _edge_w(edge_feats.T, edge_weight, W_dist, b_dist)
    partials = _sc_aggregate(x_nb, w_msg,
                             senders.astype(jnp.int32),
                             receivers.astype(jnp.int32))
    return _combine(node_feats, partials, W_comb, b_comb)


# speedup vs baseline: 1.0136x; 1.0136x over previous
import functools
import math

import jax
import jax.numpy as jnp
from jax import lax
from jax.experimental import pallas as pl
from jax.experimental.pallas import tpu as pltpu
from jax.experimental.pallas import tpu_sc as plsc

N_NODES = 10000
N_EDGES = 320000
NUM_RBF = 16
NUM_CHANNELS = 128
CUTOFF = 5.0
NUM_SPECIES = 100

NC = 2
NS = 16
NW = NC * NS
EPW = N_EDGES // NW
EB = 80
NB = EPW // EB
RPS = 624


def _emb_body(z_ref, emb_ref, o_ref):
    z = z_ref[...]
    oh = (z == lax.broadcasted_iota(jnp.int32, (z.shape[0], NUM_SPECIES), 1))
    o_ref[...] = jnp.dot(oh.astype(jnp.float32), emb_ref[...],
                         preferred_element_type=jnp.float32)


def _emb_lookup(node_z, emb_table):
    bn = 2000
    return pl.pallas_call(
        _emb_body,
        grid=(N_NODES // bn,),
        in_specs=[
            pl.BlockSpec((bn, 1), lambda i: (i, 0)),
            pl.BlockSpec((NUM_SPECIES, NUM_CHANNELS), lambda i: (0, 0)),
        ],
        out_specs=pl.BlockSpec((bn, NUM_CHANNELS), lambda i: (i, 0)),
        out_shape=jax.ShapeDtypeStruct((N_NODES, NUM_CHANNELS), jnp.float32),
    )(node_z.reshape(-1, 1), emb_table)


def _edge_w_body(eft_ref, ew_ref, wd_ref, bd_ref, o_ref):
    be = o_ref.shape[0]
    ew = ew_ref[0]
    c = 0.5 * (jnp.cos(ew * (math.pi / CUTOFF)) + 1.0)
    c = jnp.where(ew < CUTOFF, c, 0.0)
    eye = (lax.broadcasted_iota(jnp.int32, (NUM_CHANNELS, NUM_CHANNELS), 0)
           == lax.broadcasted_iota(jnp.int32, (NUM_CHANNELS, NUM_CHANNELS), 1)
           ).astype(jnp.float32)
    ct = lax.dot_general(eye, c, dimension_numbers=(((1,), (1,)), ((), ())),
                         preferred_element_type=jnp.float32)
    wd = wd_ref[...]
    bd = bd_ref[...]
    for g in range(be // NUM_CHANNELS):
        wg = lax.dot_general(
            eft_ref[:, pl.ds(NUM_CHANNELS * g, NUM_CHANNELS)], wd,
            dimension_numbers=(((0,), (0,)), ((), ())),
            preferred_element_type=jnp.float32)
        o_ref[pl.ds(NUM_CHANNELS * g, NUM_CHANNELS), :] = (
            (wg + bd) * ct[:, g:g + 1])


def _edge_w(edge_feats_t, edge_weight, w_dist, b_dist):
    be = 12800
    return pl.pallas_call(
        _edge_w_body,
        grid=(N_EDGES // be,),
        in_specs=[
            pl.BlockSpec((NUM_RBF, be), lambda i: (0, i)),
            pl.BlockSpec((1, be // NUM_CHANNELS, NUM_CHANNELS),
                         lambda i: (i, 0, 0)),
            pl.BlockSpec((NUM_RBF, NUM_CHANNELS), lambda i: (0, 0)),
            pl.BlockSpec((1, NUM_CHANNELS), lambda i: (0, 0)),
        ],
        out_specs=pl.BlockSpec((be, NUM_CHANNELS), lambda i: (i, 0)),
        out_shape=jax.ShapeDtypeStruct((N_EDGES, NUM_CHANNELS), jnp.float32),
    )(edge_feats_t,
      edge_weight.reshape(N_EDGES // be, be // NUM_CHANNELS, NUM_CHANNELS),
      w_dist, b_dist.reshape(1, -1))


NDAT = 2
NIDX = 4
NPIPE = (NB - 1) // NIDX * NIDX


def _sc_body(xnb_hbm, w_hbm, snd_hbm, rcv_hbm, out_hbm,
             s_v, r_v, w_v, x_v, acc, sem_i, sem_g, sem_w, sem_s):
    cid = lax.axis_index("c")
    sid = lax.axis_index("s")

    def _zrow(i, _):
        for j in range(NUM_CHANNELS // 16):
            w_v[0, i, pl.ds(16 * j, 16)] = jnp.zeros((16,), jnp.float32)
        return 0
    lax.fori_loop(0, EB, _zrow, 0)
    row0 = sid * RPS
    for k in range(RPS // EB):
        pltpu.sync_copy(w_v.at[0], acc.at[pl.ds(row0 + k * EB, EB)])
    rem = RPS - (RPS // EB) * EB
    pltpu.sync_copy(w_v.at[0, pl.ds(0, rem)],
                    acc.at[pl.ds(row0 + (RPS // EB) * EB, rem)])
    tail = N_NODES - NS * RPS
    @pl.when(sid == NS - 1)
    def _zero_tail():
        pltpu.sync_copy(w_v.at[0, pl.ds(0, tail)], acc.at[pl.ds(NS * RPS, tail)])
    plsc.subcore_barrier()

    ebase = (cid * NS + sid) * EPW

    def _issue_idx(blk, ki):
        base = ebase + blk * EB
        pltpu.async_copy(snd_hbm.at[pl.ds(base, EB)], s_v.at[ki], sem_i.at[ki])
        pltpu.async_copy(rcv_hbm.at[pl.ds(base, EB)], r_v.at[ki], sem_i.at[ki])

    def _wait_idx(ki):
        pltpu.make_async_copy(snd_hbm.at[pl.ds(0, EB)], s_v.at[ki], sem_i.at[ki]).wait()
        pltpu.make_async_copy(rcv_hbm.at[pl.ds(0, EB)], r_v.at[ki], sem_i.at[ki]).wait()

    def _issue_fetch(blk, kd, ki):
        base = ebase + blk * EB
        pltpu.async_copy(w_hbm.at[pl.ds(base, EB)], w_v.at[kd], sem_w.at[kd])
        pltpu.async_copy(xnb_hbm.at[s_v.at[ki]], x_v.at[kd], sem_g.at[kd])

    def _wait_fetch(kd, ki):
        pltpu.make_async_copy(w_hbm.at[pl.ds(0, EB)], w_v.at[kd], sem_w.at[kd]).wait()
        pltpu.make_async_copy(xnb_hbm.at[s_v.at[ki]], x_v.at[kd], sem_g.at[kd]).wait()

    def _wait_scatter(kd, ki):
        pltpu.make_async_copy(x_v.at[kd], acc.at[r_v.at[ki]], sem_s.at[kd]).wait()

    def _mult(kd, ki):
        def _mrow(q, _):
            for u in range(2):
                i = q * 2 + u
                for jj in range(NUM_CHANNELS // 16):
                    sl = pl.ds(16 * jj, 16)
                    x_v[kd, i, sl] = x_v[kd, i, sl] * w_v[kd, i, sl]
            return 0
        lax.fori_loop(0, EB // 2, _mrow, 0)

    pltpu.sync_copy(snd_hbm.at[pl.ds(ebase, EB)], s_v.at[0])
    pltpu.sync_copy(rcv_hbm.at[pl.ds(ebase, EB)], r_v.at[0])
    _issue_fetch(0, 0, 0)
    _issue_idx(1, 1)
    _issue_idx(2, 2)

    def _outer(m, _):
        for t in range(NIDX):
            j = m * NIDX + t
            kd = t % NDAT
            _wait_fetch(t % NDAT, t)

            @pl.when(j >= 1)
            def _drain_prev():
                _wait_scatter((t + 1) % NDAT, (t + 3) % NIDX)

            _wait_idx((t + 1) % NIDX)
            _issue_fetch(j + 1, (t + 1) % NDAT, (t + 1) % NIDX)
            _mult(kd, t)
            pltpu.async_copy(x_v.at[kd], acc.at[r_v.at[t]], sem_s.at[kd],
                             add=True)

            @pl.when(j + 3 <= NPIPE)
            def _idx_ahead():
                _issue_idx(j + 3, (t + 3) % NIDX)
        return 0

    lax.fori_loop(0, NPIPE // NIDX, _outer, 0)
    _wait_scatter((NPIPE + 1) % NDAT, (NPIPE + 3) % NIDX)
    _wait_fetch(NPIPE % NDAT, NPIPE % NIDX)
    _mult(NPIPE % NDAT, NPIPE % NIDX)
    pltpu.async_copy(x_v.at[NPIPE % NDAT], acc.at[r_v.at[NPIPE % NIDX]],
                     sem_s.at[NPIPE % NDAT], add=True)
    _wait_scatter(NPIPE % NDAT, NPIPE % NIDX)
    plsc.subcore_barrier()
    pltpu.sync_copy(acc.at[pl.ds(row0, RPS)], out_hbm.at[cid, pl.ds(row0, RPS)])
    @pl.when(sid == NS - 1)
    def _out_tail():
        pltpu.sync_copy(acc.at[pl.ds(NS * RPS, tail)],
                        out_hbm.at[cid, pl.ds(NS * RPS, tail)])


def _sc_aggregate(x_nb, w_msg, senders, receivers):
    mesh = plsc.VectorSubcoreMesh(core_axis_name="c", subcore_axis_name="s",
                                  num_cores=NC, num_subcores=NS)
    f = pl.kernel(
        _sc_body,
        out_type=jax.ShapeDtypeStruct((NC, N_NODES, NUM_CHANNELS), jnp.float32),
        mesh=mesh,
        scratch_types=[
            pltpu.VMEM((NIDX, EB), jnp.int32),
            pltpu.VMEM((NIDX, EB), jnp.int32),
            pltpu.VMEM((NDAT, EB, NUM_CHANNELS), jnp.float32),
            pltpu.VMEM((NDAT, EB, NUM_CHANNELS), jnp.float32),
            pltpu.VMEM_SHARED((N_NODES, NUM_CHANNELS), jnp.float32),
            pltpu.SemaphoreType.DMA((NIDX,)),
            pltpu.SemaphoreType.DMA((NDAT,)),
            pltpu.SemaphoreType.DMA((NDAT,)),
            pltpu.SemaphoreType.DMA((NDAT,)),
        ],
    )
    return f(x_nb, w_msg, senders, receivers)


def _combine_body(nf_ref, p_ref, w1_ref, w2_ref, b_ref, o_ref):
    agg = p_ref[0] + p_ref[1]
    o_ref[...] = (jnp.dot(nf_ref[...], w1_ref[...],
                          preferred_element_type=jnp.float32)
                  + jnp.dot(agg, w2_ref[...],
                            preferred_element_type=jnp.float32)
                  + b_ref[...])


def _combine(node_feats, partials, w_comb, b_comb):
    bn = 2000
    return pl.pallas_call(
        _combine_body,
        grid=(N_NODES // bn,),
        in_specs=[
            pl.BlockSpec((bn, NUM_CHANNELS), lambda i: (i, 0)),
            pl.BlockSpec((NC, bn, NUM_CHANNELS), lambda i: (0, i, 0)),
            pl.BlockSpec((NUM_CHANNELS, NUM_CHANNELS), lambda i: (0, 0)),
            pl.BlockSpec((NUM_CHANNELS, NUM_CHANNELS), lambda i: (0, 0)),
            pl.BlockSpec((1, NUM_CHANNELS), lambda i: (0, 0)),
        ],
        out_specs=pl.BlockSpec((bn, NUM_CHANNELS), lambda i: (i, 0)),
        out_shape=jax.ShapeDtypeStruct((N_NODES, NUM_CHANNELS), jnp.float32),
    )(node_feats, partials, w_comb[:NUM_CHANNELS], w_comb[NUM_CHANNELS:],
      b_comb.reshape(1, -1))


def kernel(node_z, node_feats, senders, receivers, edge_weight, edge_feats,
           emb_table, W_dist, b_dist, W_comb, b_comb):
    x_nb = _emb_lookup(node_z.astype(jnp.int32), emb_table)
    w_msg = _edge_w(edge_feats.T, edge_weight, W_dist, b_dist)
    partials = _sc_aggregate(x_nb, w_msg,
                             senders.astype(jnp.int32),
                             receivers.astype(jnp.int32))
    return _combine(node_feats, partials, W_comb, b_comb)
